# Initial kernel scaffold; baseline (speedup 1.0000x reference)
import jax
import jax.numpy as jnp
from jax.experimental import pallas as pl


def kernel(x, edge_index, edge_vals, W, b):
    raise NotImplementedError("write your pallas kernel here")


# speedup vs baseline: 10.5622x; 10.5622x over previous
import functools

import jax
import jax.numpy as jnp
from jax import lax
from jax.experimental import pallas as pl
from jax.experimental.pallas import tpu as pltpu
from jax.experimental.pallas import tpu_sc as plsc

NC = 2
NS = 16
L = 16
CHUNK = 80


def _sc_spmm(x, src, dst, vals, zrows):
    n, d = x.shape
    e = vals.shape[0]
    edges_per_tile = e // (NC * NS)
    nchunk = edges_per_tile // CHUNK
    r0 = (n // NS) // 8 * 8
    r_last = n - (NS - 1) * r0

    mesh = plsc.VectorSubcoreMesh(core_axis_name="c", subcore_axis_name="s")

    @functools.partial(
        pl.kernel,
        mesh=mesh,
        out_type=jax.ShapeDtypeStruct((NC * n, d), jnp.float32),
        scratch_types=[
            pltpu.VMEM((edges_per_tile,), jnp.int32),
            pltpu.VMEM((2, CHUNK), jnp.int32),
            pltpu.VMEM((edges_per_tile,), jnp.float32),
            pltpu.VMEM((CHUNK, d), jnp.float32),
            pltpu.VMEM((CHUNK, d), jnp.float32),
            pltpu.VMEM_SHARED((n, d), jnp.float32),
            pltpu.SemaphoreType.DMA,
            pltpu.SemaphoreType.DMA,
            pltpu.SemaphoreType.DMA,
            pltpu.SemaphoreType.DMA,
        ],
    )
    def spmm(x_hbm, src_hbm, dst_hbm, vals_hbm, z_hbm, out_hbm,
             src_v, dst_v, vals_v, rows_a, rows_b, acc_sh,
             sem_a, sem_b, sem_da, sem_db):
        cid = lax.axis_index("c")
        sid = lax.axis_index("s")
        wid = cid * NS + sid

        rbase = sid * r0

        @pl.when(sid < NS - 1)
        def _zero_main():
            pltpu.sync_copy(z_hbm.at[pl.ds(0, r0)],
                            acc_sh.at[pl.ds(rbase, r0)])

        @pl.when(sid == NS - 1)
        def _zero_last():
            pltpu.sync_copy(z_hbm.at[pl.ds(0, r_last)],
                            acc_sh.at[pl.ds((NS - 1) * r0, r_last)])

        ebase = wid * edges_per_tile
        pltpu.sync_copy(src_hbm.at[pl.ds(ebase, edges_per_tile)], src_v)
        pltpu.sync_copy(vals_hbm.at[pl.ds(ebase, edges_per_tile)], vals_v)

        plsc.subcore_barrier()

        def gather(c, rv, sem):
            return pltpu.make_async_copy(
                x_hbm.at[src_v.at[pl.ds(c * CHUNK, CHUNK)]], rv, sem)

        def dst_fetch(c, b, sem):
            return pltpu.make_async_copy(
                dst_hbm.at[pl.ds(ebase + c * CHUNK, CHUNK)], dst_v.at[b], sem)

        gather(0, rows_a, sem_a).start()
        gather(1, rows_b, sem_b).start()
        dst_fetch(0, 0, sem_da).start()
        dst_fetch(1, 1, sem_db).start()

        def step(i, carry):
            for b, (rv, sem, dsem) in enumerate(
                    ((rows_a, sem_a, sem_da), (rows_b, sem_b, sem_db))):
                c = i * 2 + b

                @pl.when(c < nchunk)
                def _process():
                    gather(c, rv, sem).wait()
                    dst_fetch(c, b, dsem).wait()

                    def group_body(g, carry2):
                        vals16 = vals_v[pl.ds(c * CHUNK + g * L, L)]
                        for lane in range(L):
                            vv = jnp.broadcast_to(vals16[lane], (L,))
                            k = g * L + lane
                            for j in range(d // L):
                                s = pl.ds(j * L, L)
                                rv[k, s] = rv[k, s] * vv
                        return carry2

                    lax.fori_loop(0, CHUNK // L, group_body, 0)

                    pltpu.sync_copy(rv, acc_sh.at[dst_v.at[b]], add=True)

                    @pl.when(c + 2 < nchunk)
                    def _prefetch():
                        gather(c + 2, rv, sem).start()
                        dst_fetch(c + 2, b, dsem).start()

            return carry

        lax.fori_loop(0, (nchunk + 1) // 2, step, 0)

        plsc.subcore_barrier()
        plsc.subcore_barrier()

        @pl.when(sid < NS - 1)
        def _write_main():
            pltpu.sync_copy(acc_sh.at[pl.ds(rbase, r0)],
                            out_hbm.at[pl.ds(cid * n + rbase, r0)])

        @pl.when(sid == NS - 1)
        def _write_last():
            pltpu.sync_copy(
                acc_sh.at[pl.ds((NS - 1) * r0, r_last)],
                out_hbm.at[pl.ds(cid * n + (NS - 1) * r0, r_last)])

    return spmm(x, src, dst, vals, zrows)


def _tc_combine_matmul(p0, p1, W, b2d):
    n, d_in = p0.shape
    d_out = W.shape[1]
    bm = 1000

    def body(p0_ref, p1_ref, w_ref, b_ref, o_ref):
        acc = p0_ref[...] + p1_ref[...]
        o_ref[...] = (
            jnp.dot(acc, w_ref[...], preferred_element_type=jnp.float32)
            + b_ref[...])

    return pl.pallas_call(
        body,
        grid=(n // bm,),
        in_specs=[
            pl.BlockSpec((bm, d_in), lambda i: (i, 0)),
            pl.BlockSpec((bm, d_in), lambda i: (i, 0)),
            pl.BlockSpec((d_in, d_out), lambda i: (0, 0)),
            pl.BlockSpec((1, d_out), lambda i: (0, 0)),
        ],
        out_specs=pl.BlockSpec((bm, d_out), lambda i: (i, 0)),
        out_shape=jax.ShapeDtypeStruct((n, d_out), jnp.float32),
    )(p0, p1, W, b2d)


def kernel(x, edge_index, edge_vals, W, b):
    n, d = x.shape
    e = edge_vals.shape[0]
    assert e % (NC * NS * CHUNK) == 0 and n % NS == 0 and d % L == 0

    src = edge_index[1]
    dst = edge_index[0]
    r_last = n - (NS - 1) * ((n // NS) // 8 * 8)
    zrows = jnp.zeros((r_last, d), jnp.float32)

    partials = _sc_spmm(x, src, dst, edge_vals, zrows)
    return _tc_combine_matmul(partials[:n], partials[n:], W,
                              b.reshape(1, -1))
